# Initial kernel scaffold; baseline (speedup 1.0000x reference)
import jax
import jax.numpy as jnp
from jax.experimental import pallas as pl


def kernel(idxs, emb):
    raise NotImplementedError("write your pallas kernel here")


# speedup vs baseline: 13.7002x; 13.7002x over previous
import functools

import jax
import jax.numpy as jnp
from jax import lax
from jax.experimental import pallas as pl
from jax.experimental.pallas import tpu as pltpu
from jax.experimental.pallas import tpu_sc as plsc

B = 16384
NG = 200
D = 32
NC = 2
NS = 16
NW = NC * NS
ROWS_PER_TILE = B // NW
IDX_BLOCK = 64
N_BLOCKS = ROWS_PER_TILE // IDX_BLOCK
L = 16
RED_UNROLL = 8


def _reduce_bag(rows_v, r):
    zero = jnp.zeros((L,), jnp.float32)

    def body(it, carry):
        a = list(carry)
        j0 = it * RED_UNROLL
        for c in range(RED_UNROLL):
            lo = rows_v[j0 + c, 0:L]
            hi = rows_v[j0 + c, L:D]
            k = (c % 4) * 2
            a[k] = a[k] + lo
            a[k + 1] = a[k + 1] + hi
        return tuple(a)

    acc = lax.fori_loop(0, NG // RED_UNROLL, body, (zero,) * 8)
    lo = (acc[0] + acc[2]) + (acc[4] + acc[6])
    hi = (acc[1] + acc[3]) + (acc[5] + acc[7])
    return lo, hi


def _sc_bag_sums(idxs, emb):
    mesh = plsc.VectorSubcoreMesh(core_axis_name="c", subcore_axis_name="s")

    @functools.partial(
        pl.kernel,
        out_type=jax.ShapeDtypeStruct((B, D), jnp.float32),
        mesh=mesh,
        scratch_types=[
            pltpu.VMEM((IDX_BLOCK * NG,), jnp.int32),
            pltpu.VMEM((NG, D), jnp.float32),
            pltpu.VMEM((NG, D), jnp.float32),
            pltpu.VMEM((ROWS_PER_TILE, D), jnp.float32),
            pltpu.SemaphoreType.DMA,
            pltpu.SemaphoreType.DMA,
        ],
        compiler_params=pltpu.CompilerParams(use_tc_tiling_on_sc=False),
    )
    def k(idx_hbm, emb_hbm, out_hbm, idx_v, rows0, rows1, out_v, sem0, sem1):
        wid = lax.axis_index("s") * NC + lax.axis_index("c")
        base = wid * ROWS_PER_TILE
        rows = (rows0, rows1)
        sems = (sem0, sem1)

        def gather(r, buf):
            off = pl.multiple_of(r * NG, 8)
            pltpu.async_copy(
                emb_hbm.at[idx_v.at[pl.ds(off, NG)]], rows[buf], sems[buf])

        def wait(buf):
            pltpu.make_async_copy(
                emb_hbm.at[idx_v.at[pl.ds(0, NG)]], rows[buf], sems[buf]
            ).wait()

        for blk in range(N_BLOCKS):
            pltpu.sync_copy(
                idx_hbm.at[pl.ds((base + blk * IDX_BLOCK) * NG, IDX_BLOCK * NG)],
                idx_v)
            gather(0, 0)
            gather(1, 1)

            def step(i, carry):
                for b in range(2):
                    r = i + b
                    wait(b)
                    lo, hi = _reduce_bag(rows[b], 0)
                    orow = blk * IDX_BLOCK + r
                    out_v[orow, 0:L] = lo
                    out_v[orow, L:D] = hi

                    @pl.when(r + 2 < IDX_BLOCK)
                    def _prefetch(b=b, r=r):
                        gather(r + 2, b)
                return carry

            lax.fori_loop(0, IDX_BLOCK // 2, lambda i, c: step(i * 2, c), 0)

        pltpu.sync_copy(out_v, out_hbm.at[pl.ds(base, ROWS_PER_TILE)])

    return k(idxs.reshape(B * NG), emb)


def _normalize_block(x_ref, o_ref):
    x = x_ref[...]
    norm = jnp.sqrt(jnp.sum(x * x, axis=1, keepdims=True))
    o_ref[...] = x / jnp.maximum(norm, 1e-12)


def _tc_normalize(vecs):
    blk = 2048
    return pl.pallas_call(
        _normalize_block,
        out_shape=jax.ShapeDtypeStruct((B, D), jnp.float32),
        grid=(B // blk,),
        in_specs=[pl.BlockSpec((blk, D), lambda i: (i, 0))],
        out_specs=pl.BlockSpec((blk, D), lambda i: (i, 0)),
    )(vecs)


def kernel(idxs, emb):
    return _tc_normalize(_sc_bag_sums(idxs, emb))


# speedup vs baseline: 13.7200x; 1.0015x over previous
import functools

import jax
import jax.numpy as jnp
from jax import lax
from jax.experimental import pallas as pl
from jax.experimental.pallas import tpu as pltpu
from jax.experimental.pallas import tpu_sc as plsc

B = 16384
NG = 200
D = 32
NC = 2
NS = 16
NW = NC * NS
ROWS_PER_TILE = B // NW
IDX_BLOCK = 64
N_BLOCKS = ROWS_PER_TILE // IDX_BLOCK
L = 16
RED_UNROLL = 8


def _reduce_bag(rows_v, r):
    zero = jnp.zeros((L,), jnp.float32)

    def body(it, carry):
        a = list(carry)
        j0 = it * RED_UNROLL
        for c in range(RED_UNROLL):
            lo = rows_v[j0 + c, 0:L]
            hi = rows_v[j0 + c, L:D]
            k = (c % 4) * 2
            a[k] = a[k] + lo
            a[k + 1] = a[k + 1] + hi
        return tuple(a)

    acc = lax.fori_loop(0, NG // RED_UNROLL, body, (zero,) * 8)
    lo = (acc[0] + acc[2]) + (acc[4] + acc[6])
    hi = (acc[1] + acc[3]) + (acc[5] + acc[7])
    return lo, hi


def _sc_bag_sums(idxs, emb):
    mesh = plsc.VectorSubcoreMesh(core_axis_name="c", subcore_axis_name="s")

    @functools.partial(
        pl.kernel,
        out_type=jax.ShapeDtypeStruct((B, D), jnp.float32),
        mesh=mesh,
        scratch_types=[
            pltpu.VMEM((IDX_BLOCK, NG), jnp.int32),
            pltpu.VMEM((NG, D), jnp.float32),
            pltpu.VMEM((NG, D), jnp.float32),
            pltpu.VMEM((ROWS_PER_TILE, D), jnp.float32),
            pltpu.SemaphoreType.DMA,
            pltpu.SemaphoreType.DMA,
        ],
        compiler_params=pltpu.CompilerParams(use_tc_tiling_on_sc=False),
    )
    def k(idx_hbm, emb_hbm, out_hbm, idx_v, rows0, rows1, out_v, sem0, sem1):
        wid = lax.axis_index("s") * NC + lax.axis_index("c")
        base = wid * ROWS_PER_TILE
        rows = (rows0, rows1)
        sems = (sem0, sem1)

        def gather(r, buf):
            pltpu.async_copy(
                emb_hbm.at[idx_v.at[r]], rows[buf], sems[buf])

        def wait(buf):
            pltpu.make_async_copy(
                emb_hbm.at[idx_v.at[0]], rows[buf], sems[buf]
            ).wait()

        for blk in range(N_BLOCKS):
            pltpu.sync_copy(
                idx_hbm.at[pl.ds(base + blk * IDX_BLOCK, IDX_BLOCK)],
                idx_v)
            gather(0, 0)
            gather(1, 1)

            def step(i, carry):
                for b in range(2):
                    r = i + b
                    wait(b)
                    lo, hi = _reduce_bag(rows[b], 0)
                    orow = blk * IDX_BLOCK + r
                    out_v[orow, 0:L] = lo
                    out_v[orow, L:D] = hi

                    @pl.when(r + 2 < IDX_BLOCK)
                    def _prefetch(b=b, r=r):
                        gather(r + 2, b)
                return carry

            lax.fori_loop(0, IDX_BLOCK // 2, lambda i, c: step(i * 2, c), 0)

        pltpu.sync_copy(out_v, out_hbm.at[pl.ds(base, ROWS_PER_TILE)])

    return k(idxs, emb)


def _normalize_block(x_ref, o_ref):
    x = x_ref[...]
    norm = jnp.sqrt(jnp.sum(x * x, axis=1, keepdims=True))
    o_ref[...] = x / jnp.maximum(norm, 1e-12)


def _tc_normalize(vecs):
    blk = 2048
    return pl.pallas_call(
        _normalize_block,
        out_shape=jax.ShapeDtypeStruct((B, D), jnp.float32),
        grid=(B // blk,),
        in_specs=[pl.BlockSpec((blk, D), lambda i: (i, 0))],
        out_specs=pl.BlockSpec((blk, D), lambda i: (i, 0)),
    )(vecs)


def kernel(idxs, emb):
    return _tc_normalize(_sc_bag_sums(idxs, emb))
